# Initial kernel scaffold; baseline (speedup 1.0000x reference)
import jax
import jax.numpy as jnp
from jax.experimental import pallas as pl


def kernel(x, edge_index, edge_attr, W1, b1, W2, b2, Wc1, bc1, Wc2, bc2, Wc3, bc3, Wc4, bc4):
    raise NotImplementedError("write your pallas kernel here")


# speedup vs baseline: 4.1294x; 4.1294x over previous
import functools

import jax
import jax.numpy as jnp
from jax import lax
from jax.experimental import pallas as pl
from jax.experimental.pallas import tpu as pltpu
from jax.experimental.pallas import tpu_sc as plsc

_NC = 2
_NS = 16
_C = 128
_BR = 256
_BE = 512


def _sc_mesh():
    return plsc.VectorSubcoreMesh(core_axis_name="c", subcore_axis_name="s")


def _build_deg(EP, NP):
    rpt = NP // _NS
    ept = EP // (_NC * _NS)
    nch = ept // _C

    @functools.partial(
        pl.kernel,
        out_type=jax.ShapeDtypeStruct((_NC, NP), jnp.float32),
        mesh=_sc_mesh(),
        scratch_types=[
            pltpu.VMEM((_C,), jnp.int32),
            pltpu.VMEM((_C,), jnp.float32),
            pltpu.VMEM_SHARED((NP,), jnp.float32),
        ],
    )
    def k(dst_hbm, ones_hbm, zcol_hbm, out_hbm, didx, ones_v, acc):
        c = lax.axis_index("c")
        s = lax.axis_index("s")
        r0 = s * rpt
        pltpu.sync_copy(zcol_hbm.at[pl.ds(r0, rpt)], acc.at[pl.ds(r0, rpt)])
        pltpu.sync_copy(ones_hbm, ones_v)
        plsc.subcore_barrier()
        t0 = (c * _NS + s) * ept

        def body(i, carry):
            eb = t0 + i * _C
            pltpu.sync_copy(dst_hbm.at[pl.ds(eb, _C)], didx)
            pltpu.sync_copy(ones_v, acc.at[didx], add=True)
            return carry

        lax.fori_loop(0, nch, body, 0)
        plsc.subcore_barrier()
        pltpu.sync_copy(acc.at[pl.ds(r0, rpt)], out_hbm.at[c, pl.ds(r0, rpt)])

    return k


def _build_agg_edge_split(EP, NP, F):
    rpt = NP // _NS
    ept = EP // (_NC * _NS)
    nch = ept // _C

    @functools.partial(
        pl.kernel,
        out_type=jax.ShapeDtypeStruct((_NC, NP, F), jnp.float32),
        mesh=_sc_mesh(),
        scratch_types=[
            pltpu.VMEM((_C,), jnp.int32),
            pltpu.VMEM((_C,), jnp.int32),
            pltpu.VMEM((_C, F), jnp.float32),
            pltpu.SemaphoreType.DMA,
            pltpu.VMEM_SHARED((NP, F), jnp.float32),
        ],
    )
    def k(src_hbm, dst_hbm, z_hbm, zeros_hbm, out_hbm, sidx, didx, rows, sem, acc):
        c = lax.axis_index("c")
        s = lax.axis_index("s")
        r0 = s * rpt

        @pl.when(c == 0)
        def _():
            pltpu.sync_copy(z_hbm.at[pl.ds(r0, rpt)], acc.at[pl.ds(r0, rpt)])

        @pl.when(c == 1)
        def _():
            pltpu.sync_copy(zeros_hbm.at[pl.ds(r0, rpt)], acc.at[pl.ds(r0, rpt)])

        plsc.subcore_barrier()
        t0 = (c * _NS + s) * ept

        def body(i, carry):
            eb = t0 + i * _C
            pltpu.sync_copy(src_hbm.at[pl.ds(eb, _C)], sidx)
            pltpu.sync_copy(dst_hbm.at[pl.ds(eb, _C)], didx)
            pltpu.async_copy(z_hbm.at[sidx], rows, sem).wait()
            pltpu.sync_copy(rows, acc.at[didx], add=True)
            return carry

        lax.fori_loop(0, nch, body, 0)
        plsc.subcore_barrier()
        pltpu.sync_copy(acc.at[pl.ds(r0, rpt)], out_hbm.at[c, pl.ds(r0, rpt)])

    return k


def _build_agg_feat_split(EP, NP, F):
    rpt = NP // _NS
    ept = EP // _NS
    nch = ept // _C

    @functools.partial(
        pl.kernel,
        out_type=jax.ShapeDtypeStruct((_NC, NP, F), jnp.float32),
        mesh=_sc_mesh(),
        scratch_types=[
            pltpu.VMEM((_C,), jnp.int32),
            pltpu.VMEM((_C,), jnp.int32),
            pltpu.VMEM((_C, F), jnp.float32),
            pltpu.SemaphoreType.DMA,
            pltpu.VMEM_SHARED((NP, F), jnp.float32),
        ],
    )
    def k(src_hbm, dst_hbm, za_hbm, zb_hbm, out_hbm, sidx, didx, rows, sem, acc):
        c = lax.axis_index("c")
        s = lax.axis_index("s")
        r0 = s * rpt

        def init(table):
            pltpu.sync_copy(table.at[pl.ds(r0, rpt)], acc.at[pl.ds(r0, rpt)])

        pl.when(c == 0)(lambda: init(za_hbm))
        pl.when(c == 1)(lambda: init(zb_hbm))
        plsc.subcore_barrier()
        t0 = s * ept

        def pipe(table):
            def body(i, carry):
                eb = t0 + i * _C
                pltpu.sync_copy(src_hbm.at[pl.ds(eb, _C)], sidx)
                pltpu.sync_copy(dst_hbm.at[pl.ds(eb, _C)], didx)
                pltpu.async_copy(table.at[sidx], rows, sem).wait()
                pltpu.sync_copy(rows, acc.at[didx], add=True)
                return carry

            lax.fori_loop(0, nch, body, 0)

        pl.when(c == 0)(lambda: pipe(za_hbm))
        pl.when(c == 1)(lambda: pipe(zb_hbm))
        plsc.subcore_barrier()
        pltpu.sync_copy(acc.at[pl.ds(r0, rpt)], out_hbm.at[c, pl.ds(r0, rpt)])

    return k


def _build_q(EP, NP, F):
    ept = EP // _NS
    nch = ept // _C

    @functools.partial(
        pl.kernel,
        out_type=jax.ShapeDtypeStruct((_NC, EP, F), jnp.float32),
        mesh=_sc_mesh(),
        scratch_types=[
            pltpu.VMEM((_C,), jnp.int32),
            pltpu.VMEM((_C,), jnp.int32),
            pltpu.VMEM((_C, F), jnp.float32),
            pltpu.SemaphoreType.DMA,
        ],
    )
    def k(src_hbm, dst_hbm, aa, ab, ba, bb, out_hbm, sidx, didx, rows, sem):
        c = lax.axis_index("c")
        s = lax.axis_index("s")
        t0 = s * ept

        def pipe(ta, tb):
            def body(i, carry):
                eb = t0 + i * _C
                pltpu.sync_copy(src_hbm.at[pl.ds(eb, _C)], sidx)
                pltpu.sync_copy(dst_hbm.at[pl.ds(eb, _C)], didx)
                pltpu.async_copy(ta.at[sidx], rows, sem).wait()
                pltpu.async_copy(tb.at[didx], rows, sem, add=True).wait()
                pltpu.sync_copy(rows, out_hbm.at[c, pl.ds(eb, _C)])
                return carry

            lax.fori_loop(0, nch, body, 0)

        pl.when(c == 0)(lambda: pipe(aa, ba))
        pl.when(c == 1)(lambda: pipe(ab, bb))

    return k


def _node1(x_p, W1, parts):
    NP, D = x_p.shape
    H = W1.shape[1]
    grid = (NP // _BR,)

    def body(x_ref, w_ref, p_ref, z_ref, dinv_ref):
        deg = 1.0 + p_ref[0, :] + p_ref[1, :]
        dinv = lax.rsqrt(deg)
        z = jnp.dot(x_ref[...], w_ref[...], preferred_element_type=jnp.float32)
        z_ref[...] = z * dinv[:, None]
        dinv_ref[...] = dinv[:, None]

    return pl.pallas_call(
        body,
        grid=grid,
        in_specs=[
            pl.BlockSpec((_BR, D), lambda i: (i, 0)),
            pl.BlockSpec((D, H), lambda i: (0, 0)),
            pl.BlockSpec((2, _BR), lambda i: (0, i)),
        ],
        out_specs=[
            pl.BlockSpec((_BR, H), lambda i: (i, 0)),
            pl.BlockSpec((_BR, 1), lambda i: (i, 0)),
        ],
        out_shape=[
            jax.ShapeDtypeStruct((NP, H), jnp.float32),
            jax.ShapeDtypeStruct((NP, 1), jnp.float32),
        ],
    )(x_p, W1, parts)


def _node2(agg1, dinv, b1, W2):
    _, NP, H = agg1.shape
    H2 = W2.shape[1]
    Fh = H2 // 2
    grid = (NP // _BR,)

    def body(agg_ref, dinv_ref, b_ref, w_ref, za_ref, zb_ref):
        dv = dinv_ref[...]
        h1 = jnp.maximum(dv * (agg_ref[0] + agg_ref[1]) + b_ref[...], 0.0)
        z2 = jnp.dot(h1, w_ref[...], preferred_element_type=jnp.float32) * dv
        za_ref[...] = z2[:, :Fh]
        zb_ref[...] = z2[:, Fh:]

    return pl.pallas_call(
        body,
        grid=grid,
        in_specs=[
            pl.BlockSpec((2, _BR, H), lambda i: (0, i, 0)),
            pl.BlockSpec((_BR, 1), lambda i: (i, 0)),
            pl.BlockSpec((1, H), lambda i: (0, 0)),
            pl.BlockSpec((H, H2), lambda i: (0, 0)),
        ],
        out_specs=[
            pl.BlockSpec((_BR, Fh), lambda i: (i, 0)),
            pl.BlockSpec((_BR, Fh), lambda i: (i, 0)),
        ],
        out_shape=[
            jax.ShapeDtypeStruct((NP, Fh), jnp.float32),
            jax.ShapeDtypeStruct((NP, Fh), jnp.float32),
        ],
    )(agg1, dinv, b1, W2)


def _node3(agg2, dinv, b2, Wc1a, Wc1b):
    _, NP, Fh = agg2.shape
    H2 = 2 * Fh
    K = Wc1a.shape[1]
    Kh = K // 2
    grid = (NP // _BR,)

    def body(agg_ref, dinv_ref, b_ref, wa_ref, wb_ref, aa, ab, ba, bb):
        dv = dinv_ref[...]
        cat = jnp.concatenate([agg_ref[0], agg_ref[1]], axis=1)
        h2 = jnp.maximum(dv * cat + b_ref[...], 0.0)
        A = jnp.dot(h2, wa_ref[...], preferred_element_type=jnp.float32)
        B = jnp.dot(h2, wb_ref[...], preferred_element_type=jnp.float32)
        aa[...] = A[:, :Kh]
        ab[...] = A[:, Kh:]
        ba[...] = B[:, :Kh]
        bb[...] = B[:, Kh:]

    outs = [jax.ShapeDtypeStruct((NP, Kh), jnp.float32)] * 4
    return pl.pallas_call(
        body,
        grid=grid,
        in_specs=[
            pl.BlockSpec((2, _BR, Fh), lambda i: (0, i, 0)),
            pl.BlockSpec((_BR, 1), lambda i: (i, 0)),
            pl.BlockSpec((1, H2), lambda i: (0, 0)),
            pl.BlockSpec((H2, K), lambda i: (0, 0)),
            pl.BlockSpec((H2, K), lambda i: (0, 0)),
        ],
        out_specs=[pl.BlockSpec((_BR, Kh), lambda i: (i, 0))] * 4,
        out_shape=outs,
    )(agg2, dinv, b2, Wc1a, Wc1b)


def _mlp(q, ea, Wt, bc1, Wc2, bc2, Wc3, bc3, Wc4, bc4, E):
    _, EP, Fh = q.shape
    K1 = 2 * Fh
    K2 = Wc2.shape[1]
    K3 = Wc3.shape[1]
    grid = (E // _BE,)

    def body(q_ref, ea_ref, wt_ref, b1_ref, w2_ref, b2_ref, w3_ref, b3_ref,
             w4_ref, b4_ref, o_ref):
        z = jnp.concatenate([q_ref[0], q_ref[1]], axis=1)
        eav = ea_ref[...]
        z = z + eav[:, 0:1] * wt_ref[0:1, :] + eav[:, 1:2] * wt_ref[1:2, :]
        z = jnp.maximum(z + b1_ref[...], 0.0)
        z = jnp.maximum(
            jnp.dot(z, w2_ref[...], preferred_element_type=jnp.float32)
            + b2_ref[...], 0.0)
        z = jnp.maximum(
            jnp.dot(z, w3_ref[...], preferred_element_type=jnp.float32)
            + b3_ref[...], 0.0)
        lg = (jnp.dot(z, w4_ref[...], preferred_element_type=jnp.float32)
              + b4_ref[...]) * (1.0 / 1.5)
        o_ref[...] = 1.0 / (1.0 + jnp.exp(-lg))

    return pl.pallas_call(
        body,
        grid=grid,
        in_specs=[
            pl.BlockSpec((2, _BE, Fh), lambda i: (0, i, 0)),
            pl.BlockSpec((_BE, 2), lambda i: (i, 0)),
            pl.BlockSpec((2, K1), lambda i: (0, 0)),
            pl.BlockSpec((1, K1), lambda i: (0, 0)),
            pl.BlockSpec((K1, K2), lambda i: (0, 0)),
            pl.BlockSpec((1, K2), lambda i: (0, 0)),
            pl.BlockSpec((K2, K3), lambda i: (0, 0)),
            pl.BlockSpec((1, K3), lambda i: (0, 0)),
            pl.BlockSpec((K3, 1), lambda i: (0, 0)),
            pl.BlockSpec((1, 1), lambda i: (0, 0)),
        ],
        out_specs=pl.BlockSpec((_BE, 1), lambda i: (i, 0)),
        out_shape=jax.ShapeDtypeStruct((E, 1), jnp.float32),
    )(q, ea, Wt, bc1, Wc2, bc2, Wc3, bc3, Wc4, bc4)


def kernel(x, edge_index, edge_attr, W1, b1, W2, b2, Wc1, bc1, Wc2, bc2,
           Wc3, bc3, Wc4, bc4):
    N, D = x.shape
    E = edge_index.shape[1]
    H = W1.shape[1]
    H2 = W2.shape[1]

    NP = -(-N // 2560) * 2560
    EPE = _NC * _NS * _C
    EP = -(-E // EPE) * EPE

    src = edge_index[0]
    dst = edge_index[1]
    pad_idx = jnp.int32(NP - 1)
    src_p = jnp.pad(src, (0, EP - E), constant_values=pad_idx)
    dst_p = jnp.pad(dst, (0, EP - E), constant_values=pad_idx)
    x_p = jnp.pad(x, ((0, NP - N), (0, 0)))

    ones = jnp.ones((_C,), jnp.float32)
    zcol = jnp.zeros((NP,), jnp.float32)
    zbig = jnp.zeros((NP, H), jnp.float32)

    parts = _build_deg(EP, NP)(dst_p, ones, zcol)
    z1, dinv = _node1(x_p, W1, parts)
    agg1 = _build_agg_edge_split(EP, NP, H)(src_p, dst_p, z1, zbig)
    za, zb = _node2(agg1, dinv, b1.reshape(1, -1), W2)
    agg2 = _build_agg_feat_split(EP, NP, H2 // 2)(src_p, dst_p, za, zb)
    Aa, Ab, Ba, Bb = _node3(agg2, dinv, b2.reshape(1, -1),
                            Wc1[:H2], Wc1[H2:2 * H2])
    q = _build_q(EP, NP, H2 // 2)(src_p, dst_p, Aa, Ab, Ba, Bb)
    out = _mlp(q, edge_attr[:, :2], Wc1[2 * H2:], bc1.reshape(1, -1),
               Wc2, bc2.reshape(1, -1), Wc3, bc3.reshape(1, -1),
               Wc4, bc4.reshape(1, -1), E)
    return out
